# Initial kernel scaffold; baseline (speedup 1.0000x reference)
import jax
import jax.numpy as jnp
from jax.experimental import pallas as pl


def kernel(input_feat, edge_index0, edge_index1, edge_type, edge_subg_index, rel_emb, W_out, b_out, W_in, b_in, W_g0, b_g0, W_g1, b_g1, W_fc, b_fc):
    raise NotImplementedError("write your pallas kernel here")


# speedup vs baseline: 3.4342x; 3.4342x over previous
import functools

import jax
import jax.numpy as jnp
from jax import lax
from jax.experimental import pallas as pl
from jax.experimental.pallas import tpu as pltpu
from jax.experimental.pallas import tpu_sc as plsc

CH = 128
RB = 80
NSUB = 16


def _mesh():
    return plsc.VectorSubcoreMesh(core_axis_name="c", subcore_axis_name="s")


def _tc_rel_tables(rel_emb, W_out, b_out, W_in, b_in):
    def body(re_ref, wo_ref, bo_ref, wi_ref, bi_ref, out_ref):
        re = re_ref[:]
        ro = jnp.dot(re, wo_ref[:], preferred_element_type=jnp.float32) + bo_ref[:]
        ri = jnp.dot(re, wi_ref[:], preferred_element_type=jnp.float32) + bi_ref[:]
        ones = jnp.ones((16, 1), jnp.float32)
        zpad = jnp.zeros((16, 15), jnp.float32)
        out_ref[:] = jnp.concatenate(
            [jnp.concatenate([ro, ones, zpad], axis=1),
             jnp.concatenate([ri, ones, zpad], axis=1)], axis=0)

    return pl.pallas_call(
        body, out_shape=jax.ShapeDtypeStruct((32, 48), jnp.float32),
    )(rel_emb, W_out, b_out.reshape(1, -1), W_in, b_in.reshape(1, -1))


def _tc_build_x(input_feat, nd_cat, d1_cat, n):
    B = 1000
    grid = n // B

    def body(f_ref, ndo_ref, ndi_ref, d1o_ref, d1i_ref,
             xa_ref, xb_ref, nd0_ref, ns1_ref, nd1_ref):
        ndo = ndo_ref[:]
        ndi = ndi_ref[:]
        deg_o = ndo[:, 32:33]
        deg_i = ndi[:, 32:33]
        ns0 = jnp.where(deg_o > 0, lax.rsqrt(deg_o), 0.0)
        nd0_ref[:] = jnp.where(deg_i > 0, lax.rsqrt(deg_i), 0.0)
        d1o = d1o_ref[:, 0:1]
        d1i = d1i_ref[:, 0:1]
        ns1_ref[:] = jnp.where(d1o > 0, lax.rsqrt(d1o), 0.0)
        nd1_ref[:] = jnp.where(d1i > 0, lax.rsqrt(d1i), 0.0)
        rel = (ndo[:, :32] + ndi[:, :32]) * ns0
        feat = f_ref[:] * ns0
        xa_ref[:] = feat[:, :80]
        xb_ref[:] = jnp.concatenate([feat[:, 80:], rel], axis=1)

    f32 = jnp.float32
    return pl.pallas_call(
        body,
        grid=(grid,),
        in_specs=[
            pl.BlockSpec((B, 128), lambda i: (i, 0)),
            pl.BlockSpec((B, 48), lambda i: (i, 0)),
            pl.BlockSpec((B, 48), lambda i, g=grid: (i + g, 0)),
            pl.BlockSpec((B, 16), lambda i: (i, 0)),
            pl.BlockSpec((B, 16), lambda i, g=grid: (i + g, 0)),
        ],
        out_specs=[
            pl.BlockSpec((B, 80), lambda i: (i, 0)),
            pl.BlockSpec((B, 80), lambda i: (i, 0)),
            pl.BlockSpec((B, 1), lambda i: (i, 0)),
            pl.BlockSpec((B, 1), lambda i: (i, 0)),
            pl.BlockSpec((B, 1), lambda i: (i, 0)),
        ],
        out_shape=[
            jax.ShapeDtypeStruct((n, 80), f32),
            jax.ShapeDtypeStruct((n, 80), f32),
            jax.ShapeDtypeStruct((n, 1), f32),
            jax.ShapeDtypeStruct((n, 1), f32),
            jax.ShapeDtypeStruct((n, 1), f32),
        ],
    )(input_feat, nd_cat, nd_cat, d1_cat, d1_cat)


def _tc_layer(agg_cat, ndst, nsrc_next, W, b, n, W2, H, scale_out):
    B = 1000
    grid = n // B

    def body(aa_ref, ab_ref, nd_ref, ns_ref, w_ref, b_ref, oa_ref, ob_ref):
        h = (jnp.dot(aa_ref[:], w_ref[:W2, :],
                     preferred_element_type=jnp.float32)
             + jnp.dot(ab_ref[:], w_ref[W2:, :],
                       preferred_element_type=jnp.float32))
        x = jnp.maximum(h * nd_ref[:] + b_ref[:], 0.0)
        if scale_out:
            x = x * ns_ref[:]
        oa_ref[:] = x[:, : H // 2]
        ob_ref[:] = x[:, H // 2:]

    f32 = jnp.float32
    return pl.pallas_call(
        body,
        grid=(grid,),
        in_specs=[
            pl.BlockSpec((B, W2), lambda i: (i, 0)),
            pl.BlockSpec((B, W2), lambda i, g=grid: (i + g, 0)),
            pl.BlockSpec((B, 1), lambda i: (i, 0)),
            pl.BlockSpec((B, 1), lambda i: (i, 0)),
            pl.BlockSpec((2 * W2, H), lambda i: (0, 0)),
            pl.BlockSpec((1, H), lambda i: (0, 0)),
        ],
        out_specs=[
            pl.BlockSpec((B, H // 2), lambda i: (i, 0)),
            pl.BlockSpec((B, H // 2), lambda i: (i, 0)),
        ],
        out_shape=[
            jax.ShapeDtypeStruct((n, H // 2), f32),
            jax.ShapeDtypeStruct((n, H // 2), f32),
        ],
    )(agg_cat, agg_cat, ndst, nsrc_next, W, b.reshape(1, -1))


def _tc_fc(x_cat, Wcat, bcat, n):
    B = 1000
    grid = n // B

    def body(aa_ref, ab_ref, w_ref, b_ref, oa_ref, ob_ref):
        y = (jnp.dot(aa_ref[:], w_ref[:128, :],
                     preferred_element_type=jnp.float32)
             + jnp.dot(ab_ref[:], w_ref[128:, :],
                       preferred_element_type=jnp.float32)) + b_ref[:]
        oa_ref[:] = y[:, :128]
        ob_ref[:] = y[:, 128:]

    f32 = jnp.float32
    return pl.pallas_call(
        body,
        grid=(grid,),
        in_specs=[
            pl.BlockSpec((B, 128), lambda i: (i, 0)),
            pl.BlockSpec((B, 128), lambda i, g=grid: (i + g, 0)),
            pl.BlockSpec((256, 256), lambda i: (0, 0)),
            pl.BlockSpec((1, 256), lambda i: (0, 0)),
        ],
        out_specs=[
            pl.BlockSpec((B, 128), lambda i: (i, 0)),
            pl.BlockSpec((B, 128), lambda i: (i, 0)),
        ],
        out_shape=[
            jax.ShapeDtypeStruct((n, 128), f32),
            jax.ShapeDtypeStruct((n, 128), f32),
        ],
    )(x_cat, x_cat, Wcat, bcat.reshape(1, -1))


def _sc_embed(T, etype, nodes0, nodes1, n, E):
    nchunks = E // CH
    nblocks = n // RB
    zeros48 = jnp.zeros((RB, 48), jnp.float32)
    zeros16 = jnp.zeros((RB, 16), jnp.float32)
    ones16 = jnp.ones((CH, 16), jnp.float32)

    @functools.partial(
        pl.kernel,
        out_type=(jax.ShapeDtypeStruct((2 * n, 48), jnp.float32),
                  jax.ShapeDtypeStruct((2 * n, 16), jnp.float32)),
        mesh=_mesh(),
        compiler_params=pltpu.CompilerParams(use_tc_tiling_on_sc=False),
        scratch_types=[
            pltpu.VMEM((CH,), jnp.int32),
            pltpu.VMEM((CH,), jnp.int32),
            pltpu.VMEM((CH,), jnp.int32),
            pltpu.VMEM((CH,), jnp.int32),
            pltpu.VMEM((CH, 48), jnp.float32),
            pltpu.VMEM((CH, 16), jnp.float32),
            pltpu.VMEM_SHARED((n, 48), jnp.float32),
            pltpu.VMEM_SHARED((n, 16), jnp.float32),
            pltpu.SemaphoreType.DMA,
        ],
    )
    def k(T_h, et_h, n0_h, n1_h, z48_h, z16_h, o16_h, outA_h, outB_h,
          tbuf, tbuf2, nbuf, n1buf, rows, onesb, accA, accB, sem):
        c = lax.axis_index("c")
        s = lax.axis_index("s")

        @pl.loop(s, nblocks, step=NSUB)
        def _(b):
            pltpu.sync_copy(z48_h, accA.at[pl.ds(b * RB, RB)])
            pltpu.sync_copy(z16_h, accB.at[pl.ds(b * RB, RB)])

        pltpu.sync_copy(o16_h, onesb)
        plsc.subcore_barrier()

        eoff = c * E
        toff = c * 16

        @pl.loop(s, nchunks, step=NSUB)
        def _(kk):
            base = kk * CH
            pltpu.sync_copy(et_h.at[pl.ds(base, CH)], tbuf)
            pltpu.sync_copy(n0_h.at[pl.ds(eoff + base, CH)], nbuf)
            pltpu.sync_copy(n1_h.at[pl.ds(eoff + base, CH)], n1buf)

            @pl.loop(0, CH // 16)
            def _(j):
                tbuf2[pl.ds(j * 16, 16)] = tbuf[pl.ds(j * 16, 16)] + toff

            pltpu.async_copy(T_h.at[tbuf2], rows, sem).wait()
            pltpu.sync_copy(rows, accA.at[nbuf], add=True)
            pltpu.sync_copy(onesb, accB.at[n1buf], add=True)

        plsc.subcore_barrier()
        noff = c * n

        @pl.loop(s, nblocks, step=NSUB)
        def _(b):
            pltpu.sync_copy(accA.at[pl.ds(b * RB, RB)],
                            outA_h.at[pl.ds(noff + b * RB, RB)])
            pltpu.sync_copy(accB.at[pl.ds(b * RB, RB)],
                            outB_h.at[pl.ds(noff + b * RB, RB)])

    return k(T, etype, nodes0, nodes1, zeros48, zeros16, ones16)


def _sc_conv(xs_cat, src, dst, n, E, W2):
    nchunks = E // CH
    nblocks = n // RB
    zerosW = jnp.zeros((RB, W2), jnp.float32)

    @functools.partial(
        pl.kernel,
        out_type=jax.ShapeDtypeStruct((2 * n, W2), jnp.float32),
        mesh=_mesh(),
        compiler_params=pltpu.CompilerParams(use_tc_tiling_on_sc=False),
        scratch_types=[
            pltpu.VMEM((CH,), jnp.int32),
            pltpu.VMEM((CH,), jnp.int32),
            pltpu.VMEM((CH,), jnp.int32),
            pltpu.VMEM((CH, W2), jnp.float32),
            pltpu.VMEM_SHARED((n, W2), jnp.float32),
            pltpu.SemaphoreType.DMA,
        ],
    )
    def k(xs_h, src_h, dst_h, zW_h, out_h, sbuf, sbuf2, dbuf, rows, acc, sem):
        c = lax.axis_index("c")
        s = lax.axis_index("s")

        @pl.loop(s, nblocks, step=NSUB)
        def _(b):
            pltpu.sync_copy(zW_h, acc.at[pl.ds(b * RB, RB)])

        plsc.subcore_barrier()
        noff = c * n

        @pl.loop(s, nchunks, step=NSUB)
        def _(kk):
            base = kk * CH
            pltpu.sync_copy(src_h.at[pl.ds(base, CH)], sbuf)
            pltpu.sync_copy(dst_h.at[pl.ds(base, CH)], dbuf)

            @pl.loop(0, CH // 16)
            def _(j):
                sbuf2[pl.ds(j * 16, 16)] = sbuf[pl.ds(j * 16, 16)] + noff

            pltpu.async_copy(xs_h.at[sbuf2], rows, sem).wait()
            pltpu.sync_copy(rows, acc.at[dbuf], add=True)

        plsc.subcore_barrier()

        @pl.loop(s, nblocks, step=NSUB)
        def _(b):
            pltpu.sync_copy(acc.at[pl.ds(b * RB, RB)],
                            out_h.at[pl.ds(noff + b * RB, RB)])

    return k(xs_cat, src, dst, zerosW)


def _sc_final(Ycat, ssrc, sdst, n, ES):
    nchunks = ES // CH

    @functools.partial(
        pl.kernel,
        out_type=jax.ShapeDtypeStruct((ES, 128), jnp.float32),
        mesh=_mesh(),
        compiler_params=pltpu.CompilerParams(use_tc_tiling_on_sc=False),
        scratch_types=[
            pltpu.VMEM((CH,), jnp.int32),
            pltpu.VMEM((CH,), jnp.int32),
            pltpu.VMEM((CH, 128), jnp.float32),
            pltpu.VMEM((CH, 128), jnp.float32),
            pltpu.SemaphoreType.DMA,
        ],
    )
    def k(Y_h, s_h, d_h, out_h, abuf, bbuf, A, B, sem):
        c = lax.axis_index("c")
        s = lax.axis_index("s")
        wid = s * 2 + c

        @pl.loop(wid, nchunks, step=2 * NSUB)
        def _(kk):
            base = kk * CH
            pltpu.sync_copy(s_h.at[pl.ds(base, CH)], abuf)
            pltpu.sync_copy(d_h.at[pl.ds(base, CH)], bbuf)

            @pl.loop(0, CH // 16)
            def _(j):
                bbuf[pl.ds(j * 16, 16)] = bbuf[pl.ds(j * 16, 16)] + n

            pltpu.async_copy(Y_h.at[abuf], A, sem).wait()
            pltpu.async_copy(Y_h.at[bbuf], B, sem).wait()

            @pl.loop(0, CH)
            def _(r):
                @pl.loop(0, 8)
                def _(j):
                    A[r, pl.ds(j * 16, 16)] = (A[r, pl.ds(j * 16, 16)]
                                               + B[r, pl.ds(j * 16, 16)])

            pltpu.sync_copy(A, out_h.at[pl.ds(base, CH)])

    return k(Ycat, ssrc, sdst)


def kernel(input_feat, edge_index0, edge_index1, edge_type, edge_subg_index,
           rel_emb, W_out, b_out, W_in, b_in, W_g0, b_g0, W_g1, b_g1,
           W_fc, b_fc):
    n = input_feat.shape[0]
    E = edge_type.shape[0]
    ES = edge_subg_index.shape[1]

    T = _tc_rel_tables(rel_emb, W_out, b_out, W_in, b_in)
    nodes0 = edge_index0.reshape(-1)
    nodes1 = edge_index1.reshape(-1)
    nd_cat, d1_cat = _sc_embed(T, edge_type, nodes0, nodes1, n, E)

    xa, xb, nd0, ns1, nd1 = _tc_build_x(input_feat, nd_cat, d1_cat, n)
    xs_cat = jnp.concatenate([xa, xb], axis=0)

    agg0 = _sc_conv(xs_cat, edge_index0[0], edge_index0[1], n, E, 80)
    x1a, x1b = _tc_layer(agg0, nd0, ns1, W_g0, b_g0, n, 80, 256, True)
    x1s_cat = jnp.concatenate([x1a, x1b], axis=0)

    agg1 = _sc_conv(x1s_cat, edge_index1[0], edge_index1[1], n, E, 128)
    x2a, x2b = _tc_layer(agg1, nd1, nd1, W_g1, b_g1, n, 128, 256, False)
    x2_cat = jnp.concatenate([x2a, x2b], axis=0)

    Wcat = jnp.concatenate([W_fc[:256], W_fc[256:]], axis=1)
    bcat = jnp.concatenate([b_fc, jnp.zeros_like(b_fc)], axis=0)
    Yt, Yb = _tc_fc(x2_cat, Wcat, bcat, n)
    Ycat = jnp.concatenate([Yt, Yb], axis=0)

    return _sc_final(Ycat, edge_subg_index[0], edge_subg_index[1], n, ES)


# speedup vs baseline: 4.3078x; 1.2544x over previous
import functools

import jax
import jax.numpy as jnp
from jax import lax
from jax.experimental import pallas as pl
from jax.experimental.pallas import tpu as pltpu
from jax.experimental.pallas import tpu_sc as plsc

CH = 128
NB = 4
RB = 80
NSUB = 16


def _mesh():
    return plsc.VectorSubcoreMesh(core_axis_name="c", subcore_axis_name="s")


_SC_PARAMS = pltpu.CompilerParams(use_tc_tiling_on_sc=False)


def _tc_rel_tables(rel_emb, W_out, b_out, W_in, b_in):
    def body(re_ref, wo_ref, bo_ref, wi_ref, bi_ref, out_ref):
        re = re_ref[:]
        ro = jnp.dot(re, wo_ref[:], preferred_element_type=jnp.float32) + bo_ref[:]
        ri = jnp.dot(re, wi_ref[:], preferred_element_type=jnp.float32) + bi_ref[:]
        ones = jnp.ones((16, 1), jnp.float32)
        zpad = jnp.zeros((16, 15), jnp.float32)
        out_ref[:] = jnp.concatenate(
            [jnp.concatenate([ro, ones, zpad], axis=1),
             jnp.concatenate([ri, ones, zpad], axis=1)], axis=0)

    return pl.pallas_call(
        body, out_shape=jax.ShapeDtypeStruct((32, 48), jnp.float32),
    )(rel_emb, W_out, b_out.reshape(1, -1), W_in, b_in.reshape(1, -1))


def _tc_build_x(input_feat, nd_cat, d1_cat, n):
    B = 1000
    grid = n // B

    def body(f_ref, ndo_ref, ndi_ref, d1o_ref, d1i_ref,
             xa_ref, xb_ref, nd0_ref, ns1_ref, nd1_ref):
        ndo = ndo_ref[:]
        ndi = ndi_ref[:]
        deg_o = ndo[:, 32:33]
        deg_i = ndi[:, 32:33]
        ns0 = jnp.where(deg_o > 0, lax.rsqrt(deg_o), 0.0)
        nd0_ref[:] = jnp.where(deg_i > 0, lax.rsqrt(deg_i), 0.0)
        d1o = d1o_ref[:, 0:1]
        d1i = d1i_ref[:, 0:1]
        ns1_ref[:] = jnp.where(d1o > 0, lax.rsqrt(d1o), 0.0)
        nd1_ref[:] = jnp.where(d1i > 0, lax.rsqrt(d1i), 0.0)
        rel = (ndo[:, :32] + ndi[:, :32]) * ns0
        feat = f_ref[:] * ns0
        xa_ref[:] = feat[:, :80]
        xb_ref[:] = jnp.concatenate([feat[:, 80:], rel], axis=1)

    f32 = jnp.float32
    return pl.pallas_call(
        body,
        grid=(grid,),
        in_specs=[
            pl.BlockSpec((B, 128), lambda i: (i, 0)),
            pl.BlockSpec((B, 48), lambda i: (i, 0)),
            pl.BlockSpec((B, 48), lambda i, g=grid: (i + g, 0)),
            pl.BlockSpec((B, 16), lambda i: (i, 0)),
            pl.BlockSpec((B, 16), lambda i, g=grid: (i + g, 0)),
        ],
        out_specs=[
            pl.BlockSpec((B, 80), lambda i: (i, 0)),
            pl.BlockSpec((B, 80), lambda i: (i, 0)),
            pl.BlockSpec((B, 1), lambda i: (i, 0)),
            pl.BlockSpec((B, 1), lambda i: (i, 0)),
            pl.BlockSpec((B, 1), lambda i: (i, 0)),
        ],
        out_shape=[
            jax.ShapeDtypeStruct((n, 80), f32),
            jax.ShapeDtypeStruct((n, 80), f32),
            jax.ShapeDtypeStruct((n, 1), f32),
            jax.ShapeDtypeStruct((n, 1), f32),
            jax.ShapeDtypeStruct((n, 1), f32),
        ],
    )(input_feat, nd_cat, nd_cat, d1_cat, d1_cat)


def _tc_layer(agg_cat, ndst, nsrc_next, W, b, n, W2, H, scale_out):
    B = 1000
    grid = n // B

    def body(aa_ref, ab_ref, nd_ref, ns_ref, w_ref, b_ref, oa_ref, ob_ref):
        h = (jnp.dot(aa_ref[:], w_ref[:W2, :],
                     preferred_element_type=jnp.float32)
             + jnp.dot(ab_ref[:], w_ref[W2:, :],
                       preferred_element_type=jnp.float32))
        x = jnp.maximum(h * nd_ref[:] + b_ref[:], 0.0)
        if scale_out:
            x = x * ns_ref[:]
        oa_ref[:] = x[:, : H // 2]
        ob_ref[:] = x[:, H // 2:]

    f32 = jnp.float32
    return pl.pallas_call(
        body,
        grid=(grid,),
        in_specs=[
            pl.BlockSpec((B, W2), lambda i: (i, 0)),
            pl.BlockSpec((B, W2), lambda i, g=grid: (i + g, 0)),
            pl.BlockSpec((B, 1), lambda i: (i, 0)),
            pl.BlockSpec((B, 1), lambda i: (i, 0)),
            pl.BlockSpec((2 * W2, H), lambda i: (0, 0)),
            pl.BlockSpec((1, H), lambda i: (0, 0)),
        ],
        out_specs=[
            pl.BlockSpec((B, H // 2), lambda i: (i, 0)),
            pl.BlockSpec((B, H // 2), lambda i: (i, 0)),
        ],
        out_shape=[
            jax.ShapeDtypeStruct((n, H // 2), f32),
            jax.ShapeDtypeStruct((n, H // 2), f32),
        ],
    )(agg_cat, agg_cat, ndst, nsrc_next, W, b.reshape(1, -1))


def _tc_fc(x_cat, Wcat, bcat, n):
    B = 1000
    grid = n // B

    def body(aa_ref, ab_ref, w_ref, b_ref, oa_ref, ob_ref):
        y = (jnp.dot(aa_ref[:], w_ref[:128, :],
                     preferred_element_type=jnp.float32)
             + jnp.dot(ab_ref[:], w_ref[128:, :],
                       preferred_element_type=jnp.float32)) + b_ref[:]
        oa_ref[:] = y[:, :128]
        ob_ref[:] = y[:, 128:]

    f32 = jnp.float32
    return pl.pallas_call(
        body,
        grid=(grid,),
        in_specs=[
            pl.BlockSpec((B, 128), lambda i: (i, 0)),
            pl.BlockSpec((B, 128), lambda i, g=grid: (i + g, 0)),
            pl.BlockSpec((256, 256), lambda i: (0, 0)),
            pl.BlockSpec((1, 256), lambda i: (0, 0)),
        ],
        out_specs=[
            pl.BlockSpec((B, 128), lambda i: (i, 0)),
            pl.BlockSpec((B, 128), lambda i: (i, 0)),
        ],
        out_shape=[
            jax.ShapeDtypeStruct((n, 128), f32),
            jax.ShapeDtypeStruct((n, 128), f32),
        ],
    )(x_cat, x_cat, Wcat, bcat.reshape(1, -1))


def _sc_embed(T, etype, nodes0, nodes1, n, E):
    nblk = E // (CH * NB)
    nzb = n // RB
    zeros48 = jnp.zeros((RB, 48), jnp.float32)
    zeros16 = jnp.zeros((RB, 16), jnp.float32)
    ones16 = jnp.ones((CH, 16), jnp.float32)

    @functools.partial(
        pl.kernel,
        out_type=(jax.ShapeDtypeStruct((2 * n, 48), jnp.float32),
                  jax.ShapeDtypeStruct((2 * n, 16), jnp.float32)),
        mesh=_mesh(),
        compiler_params=_SC_PARAMS,
        scratch_types=[
            [pltpu.VMEM((CH,), jnp.int32) for _ in range(NB)],
            [pltpu.VMEM((CH,), jnp.int32) for _ in range(NB)],
            [pltpu.VMEM((CH,), jnp.int32) for _ in range(NB)],
            pltpu.VMEM((NB, CH, 48), jnp.float32),
            pltpu.VMEM((CH, 16), jnp.float32),
            pltpu.VMEM_SHARED((n, 48), jnp.float32),
            pltpu.VMEM_SHARED((n, 16), jnp.float32),
            pltpu.SemaphoreType.DMA((3, NB)),
            pltpu.SemaphoreType.DMA((NB,)),
            pltpu.SemaphoreType.DMA((NB,)),
            pltpu.SemaphoreType.DMA((NB,)),
        ],
    )
    def k(T_h, et_h, n0_h, n1_h, z48_h, z16_h, o16_h, outA_h, outB_h,
          tbufs, nbufs, n1bufs, rows, onesb, accA, accB,
          semI, semG, semS, semO):
        c = lax.axis_index("c")
        s = lax.axis_index("s")

        @pl.loop(s, nzb, step=NSUB)
        def _(b):
            pltpu.sync_copy(z48_h, accA.at[pl.ds(b * RB, RB)])
            pltpu.sync_copy(z16_h, accB.at[pl.ds(b * RB, RB)])

        pltpu.sync_copy(o16_h, onesb)
        plsc.subcore_barrier()

        eoff = c * E
        toff = c * 16

        @pl.loop(s, nblk, step=NSUB)
        def _(blk):
            base = blk * (NB * CH)
            ld = []
            for j in range(NB):
                ld.append(pltpu.async_copy(
                    et_h.at[pl.ds(base + j * CH, CH)], tbufs[j],
                    semI.at[0, j]))
                ld.append(pltpu.async_copy(
                    n0_h.at[pl.ds(eoff + base + j * CH, CH)], nbufs[j],
                    semI.at[1, j]))
                ld.append(pltpu.async_copy(
                    n1_h.at[pl.ds(eoff + base + j * CH, CH)], n1bufs[j],
                    semI.at[2, j]))
            for d in ld:
                d.wait()
            for j in range(NB):
                for i in range(CH // 16):
                    tbufs[j][pl.ds(i * 16, 16)] = (
                        tbufs[j][pl.ds(i * 16, 16)] + toff)
            gd = [pltpu.async_copy(T_h.at[tbufs[j]], rows.at[j], semG.at[j])
                  for j in range(NB)]
            sd = []
            for j in range(NB):
                gd[j].wait()
                sd.append(pltpu.async_copy(rows.at[j], accA.at[nbufs[j]],
                                           semS.at[j], add=True))
                sd.append(pltpu.async_copy(onesb, accB.at[n1bufs[j]],
                                           semO.at[j], add=True))
            for d in sd:
                d.wait()

        plsc.subcore_barrier()
        noff = c * n

        @pl.loop(s, nzb, step=NSUB)
        def _(b):
            pltpu.sync_copy(accA.at[pl.ds(b * RB, RB)],
                            outA_h.at[pl.ds(noff + b * RB, RB)])
            pltpu.sync_copy(accB.at[pl.ds(b * RB, RB)],
                            outB_h.at[pl.ds(noff + b * RB, RB)])

    return k(T, etype, nodes0, nodes1, zeros48, zeros16, ones16)


def _sc_conv(xs_cat, src, dst, n, E, W2, NB=NB):
    nblk = E // (CH * NB)
    nzb = n // RB
    zerosW = jnp.zeros((RB, W2), jnp.float32)

    @functools.partial(
        pl.kernel,
        out_type=jax.ShapeDtypeStruct((2 * n, W2), jnp.float32),
        mesh=_mesh(),
        compiler_params=_SC_PARAMS,
        scratch_types=[
            [pltpu.VMEM((CH,), jnp.int32) for _ in range(NB)],
            [pltpu.VMEM((CH,), jnp.int32) for _ in range(NB)],
            pltpu.VMEM((NB, CH, W2), jnp.float32),
            pltpu.VMEM_SHARED((n, W2), jnp.float32),
            pltpu.SemaphoreType.DMA((2, NB)),
            pltpu.SemaphoreType.DMA((NB,)),
            pltpu.SemaphoreType.DMA((NB,)),
        ],
    )
    def k(xs_h, src_h, dst_h, zW_h, out_h, sbufs, dbufs, rows, acc,
          semI, semG, semS):
        c = lax.axis_index("c")
        s = lax.axis_index("s")

        @pl.loop(s, nzb, step=NSUB)
        def _(b):
            pltpu.sync_copy(zW_h, acc.at[pl.ds(b * RB, RB)])

        plsc.subcore_barrier()
        noff = c * n

        @pl.loop(s, nblk, step=NSUB)
        def _(blk):
            base = blk * (NB * CH)
            ld = []
            for j in range(NB):
                ld.append(pltpu.async_copy(
                    src_h.at[pl.ds(base + j * CH, CH)], sbufs[j],
                    semI.at[0, j]))
                ld.append(pltpu.async_copy(
                    dst_h.at[pl.ds(base + j * CH, CH)], dbufs[j],
                    semI.at[1, j]))
            for d in ld:
                d.wait()
            for j in range(NB):
                for i in range(CH // 16):
                    sbufs[j][pl.ds(i * 16, 16)] = (
                        sbufs[j][pl.ds(i * 16, 16)] + noff)
            gd = [pltpu.async_copy(xs_h.at[sbufs[j]], rows.at[j], semG.at[j])
                  for j in range(NB)]
            sd = []
            for j in range(NB):
                gd[j].wait()
                sd.append(pltpu.async_copy(rows.at[j], acc.at[dbufs[j]],
                                           semS.at[j], add=True))
            for d in sd:
                d.wait()

        plsc.subcore_barrier()

        @pl.loop(s, nzb, step=NSUB)
        def _(b):
            pltpu.sync_copy(acc.at[pl.ds(b * RB, RB)],
                            out_h.at[pl.ds(noff + b * RB, RB)])

    return k(xs_cat, src, dst, zerosW)


def _sc_final(Ycat, ssrc, sdst, n, ES):
    NF = 2
    nblk = ES // (CH * NF)

    @functools.partial(
        pl.kernel,
        out_type=jax.ShapeDtypeStruct((ES, 128), jnp.float32),
        mesh=_mesh(),
        compiler_params=_SC_PARAMS,
        scratch_types=[
            [pltpu.VMEM((CH,), jnp.int32) for _ in range(NF)],
            [pltpu.VMEM((CH,), jnp.int32) for _ in range(NF)],
            pltpu.VMEM((NF, CH, 128), jnp.float32),
            pltpu.VMEM((NF, CH, 128), jnp.float32),
            pltpu.SemaphoreType.DMA((2, NF)),
            pltpu.SemaphoreType.DMA((NF,)),
            pltpu.SemaphoreType.DMA((NF,)),
            pltpu.SemaphoreType.DMA((NF,)),
        ],
    )
    def k(Y_h, s_h, d_h, out_h, abufs, bbufs, A, B, semI, semA, semB, semS):
        c = lax.axis_index("c")
        s = lax.axis_index("s")
        wid = s * 2 + c

        @pl.loop(wid, nblk, step=2 * NSUB)
        def _(blk):
            base = blk * (NF * CH)
            ld = []
            for j in range(NF):
                ld.append(pltpu.async_copy(
                    s_h.at[pl.ds(base + j * CH, CH)], abufs[j],
                    semI.at[0, j]))
                ld.append(pltpu.async_copy(
                    d_h.at[pl.ds(base + j * CH, CH)], bbufs[j],
                    semI.at[1, j]))
            for d in ld:
                d.wait()
            for j in range(NF):
                for i in range(CH // 16):
                    bbufs[j][pl.ds(i * 16, 16)] = (
                        bbufs[j][pl.ds(i * 16, 16)] + n)
            gda = [pltpu.async_copy(Y_h.at[abufs[j]], A.at[j], semA.at[j])
                   for j in range(NF)]
            gdb = [pltpu.async_copy(Y_h.at[bbufs[j]], B.at[j], semB.at[j])
                   for j in range(NF)]
            sd = []
            for j in range(NF):
                gda[j].wait()
                gdb[j].wait()

                @pl.loop(0, CH)
                def _(r):
                    for i in range(8):
                        A[j, r, pl.ds(i * 16, 16)] = (
                            A[j, r, pl.ds(i * 16, 16)]
                            + B[j, r, pl.ds(i * 16, 16)])

                sd.append(pltpu.async_copy(
                    A.at[j], out_h.at[pl.ds(base + j * CH, CH)], semS.at[j]))
            for d in sd:
                d.wait()

    return k(Ycat, ssrc, sdst)


def kernel(input_feat, edge_index0, edge_index1, edge_type, edge_subg_index,
           rel_emb, W_out, b_out, W_in, b_in, W_g0, b_g0, W_g1, b_g1,
           W_fc, b_fc):
    n = input_feat.shape[0]
    E = edge_type.shape[0]
    ES = edge_subg_index.shape[1]

    T = _tc_rel_tables(rel_emb, W_out, b_out, W_in, b_in)
    nodes0 = edge_index0.reshape(-1)
    nodes1 = edge_index1.reshape(-1)
    nd_cat, d1_cat = _sc_embed(T, edge_type, nodes0, nodes1, n, E)

    xa, xb, nd0, ns1, nd1 = _tc_build_x(input_feat, nd_cat, d1_cat, n)
    xs_cat = jnp.concatenate([xa, xb], axis=0)

    agg0 = _sc_conv(xs_cat, edge_index0[0], edge_index0[1], n, E, 80)
    x1a, x1b = _tc_layer(agg0, nd0, ns1, W_g0, b_g0, n, 80, 256, True)
    x1s_cat = jnp.concatenate([x1a, x1b], axis=0)

    agg1 = _sc_conv(x1s_cat, edge_index1[0], edge_index1[1], n, E, 128, NB=2)
    x2a, x2b = _tc_layer(agg1, nd1, nd1, W_g1, b_g1, n, 128, 256, False)
    x2_cat = jnp.concatenate([x2a, x2b], axis=0)

    Wcat = jnp.concatenate([W_fc[:256], W_fc[256:]], axis=1)
    bcat = jnp.concatenate([b_fc, jnp.zeros_like(b_fc)], axis=0)
    Yt, Yb = _tc_fc(x2_cat, Wcat, bcat, n)
    Ycat = jnp.concatenate([Yt, Yb], axis=0)

    return _sc_final(Ycat, edge_subg_index[0], edge_subg_index[1], n, ES)
